# baseline (speedup 1.0000x reference)
import functools
import math

import jax
import jax.numpy as jnp
from jax import lax
from jax.experimental import pallas as pl
from jax.experimental.pallas import tpu as pltpu
from jax.experimental.pallas import tpu_sc as plsc

_B = 16384
_NEWS_D = 768
_CAT_D = 128
_FEAT = 1024
_OUT_D = 256

_NC = 2
_NS = 16
_NW = _NC * _NS

_SLICES = (4096, 6144, 6144)
_BM = 512


def _make_sc_gather(lo, sb):
    bpw = sb // _NW
    nch = -(-bpw // 128)
    ch = bpw // nch

    @functools.partial(
        pl.kernel,
        out_type=[
            jax.ShapeDtypeStruct((sb, _NEWS_D), jnp.float32),
            jax.ShapeDtypeStruct((sb, _CAT_D), jnp.float32),
            jax.ShapeDtypeStruct((sb, _CAT_D), jnp.float32),
        ],
        mesh=plsc.VectorSubcoreMesh(core_axis_name="c", subcore_axis_name="s"),
        scratch_types=[
            pltpu.VMEM((bpw,), jnp.int32),
            pltpu.VMEM((bpw,), jnp.int32),
            pltpu.VMEM((bpw,), jnp.int32),
            pltpu.VMEM((ch, _NEWS_D), jnp.float32),
            pltpu.VMEM((ch, _CAT_D), jnp.float32),
            pltpu.SemaphoreType.DMA,
            pltpu.SemaphoreType.DMA,
        ],
    )
    def sc_gather(news_table_h, cat_table_h, sub_table_h, nid_h, cid_h, sid_h,
                  news_out, cat_out, sub_out,
                  nid_v, cid_v, sid_v, nb, cb, nsem, csem):
        wid = lax.axis_index("s") * _NC + lax.axis_index("c")
        base = lo + wid * bpw
        obase = wid * bpw
        pltpu.sync_copy(nid_h.at[pl.ds(base, bpw)], nid_v)
        pltpu.sync_copy(cid_h.at[pl.ds(base, bpw)], cid_v)
        pltpu.sync_copy(sid_h.at[pl.ds(base, bpw)], sid_v)

        def nfire(c):
            pltpu.async_copy(
                news_table_h.at[nid_v.at[pl.ds(c * ch, ch)]], nb, nsem)

        def small(table_h, idx_v, out_h, c):
            pltpu.async_copy(
                table_h.at[idx_v.at[pl.ds(c * ch, ch)]], cb, csem)
            pltpu.make_async_copy(
                table_h.at[idx_v.at[pl.ds(c * ch, ch)]], cb, csem).wait()
            pltpu.sync_copy(cb, out_h.at[pl.ds(obase + c * ch, ch)])

        for c in range(nch):
            nfire(c)
            small(cat_table_h, cid_v, cat_out, c)
            small(sub_table_h, sid_v, sub_out, c)
            pltpu.make_async_copy(
                news_table_h.at[nid_v.at[pl.ds(c * ch, ch)]], nb, nsem).wait()
            pltpu.sync_copy(nb, news_out.at[pl.ds(obase + c * ch, ch)])

    return sc_gather


_SC_GATHERS = []
_OFFS = []
_off = 0
for _sb in _SLICES:
    _SC_GATHERS.append(_make_sc_gather(_off, _sb))
    _OFFS.append(_off)
    _off += _sb


def _gelu_tanh(x):
    c0 = math.sqrt(2.0 / math.pi)
    return 0.5 * x * (1.0 + jnp.tanh(c0 * (x + 0.044715 * x * x * x)))


def _tc_compute(n_ref, c_ref, s_ref, w_ref, b_ref, o_ref):
    w = w_ref[...]
    acc = jnp.dot(n_ref[...], w[:_NEWS_D], preferred_element_type=jnp.float32)
    acc = acc + jnp.dot(c_ref[...], w[_NEWS_D:_NEWS_D + _CAT_D],
                        preferred_element_type=jnp.float32)
    acc = acc + jnp.dot(s_ref[...], w[_NEWS_D + _CAT_D:],
                        preferred_element_type=jnp.float32)
    acc = acc + b_ref[...]
    o_ref[...] = _gelu_tanh(acc)


def _tc_body_first(n_ref, c_ref, s_ref, w_ref, b_ref, o_ref):
    _tc_compute(n_ref, c_ref, s_ref, w_ref, b_ref, o_ref)


def _tc_body_next(p_ref, n_ref, c_ref, s_ref, w_ref, b_ref, o_ref):
    del p_ref
    _tc_compute(n_ref, c_ref, s_ref, w_ref, b_ref, o_ref)


def _tc_fused(lo, sb, prev, news_g, cat_g, sub_g, W, b2):
    blk0 = lo // _BM
    data_specs = [
        pl.BlockSpec((_BM, _NEWS_D), lambda i: (i, 0)),
        pl.BlockSpec((_BM, _CAT_D), lambda i: (i, 0)),
        pl.BlockSpec((_BM, _CAT_D), lambda i: (i, 0)),
        pl.BlockSpec((_FEAT, _OUT_D), lambda i: (0, 0)),
        pl.BlockSpec((1, _OUT_D), lambda i: (0, 0)),
    ]
    if prev is None:
        body, in_specs, aliases, args = (
            _tc_body_first, data_specs, {}, ())
    else:
        body = _tc_body_next
        in_specs = [pl.BlockSpec(memory_space=pl.ANY)] + data_specs
        aliases = {0: 0}
        args = (prev,)
    return pl.pallas_call(
        body,
        grid=(sb // _BM,),
        in_specs=in_specs,
        out_specs=pl.BlockSpec((_BM, _OUT_D), lambda i, _b=blk0: (_b + i, 0)),
        out_shape=jax.ShapeDtypeStruct((_B, _OUT_D), jnp.float32),
        input_output_aliases=aliases,
        compiler_params=pltpu.CompilerParams(
            dimension_semantics=("arbitrary",)),
    )(*args, news_g, cat_g, sub_g, W, b2)


def kernel(news_ids, news_categ, news_subcateg, news_table, cat_table,
           subcat_table, W, b):
    nid = news_ids.astype(jnp.int32)
    cid = news_categ.astype(jnp.int32)
    sid = news_subcateg.astype(jnp.int32)
    b2 = b.reshape(1, _OUT_D)
    gathered = [
        sc(news_table, cat_table, subcat_table, nid, cid, sid)
        for sc in _SC_GATHERS
    ]
    out = None
    for s, sb in enumerate(_SLICES):
        news_g, cat_g, sub_g = gathered[s]
        out = _tc_fused(_OFFS[s], sb, out, news_g, cat_g, sub_g, W, b2)
    return out


# speedup vs baseline: 1.0208x; 1.0208x over previous
import functools
import math

import jax
import jax.numpy as jnp
from jax import lax
from jax.experimental import pallas as pl
from jax.experimental.pallas import tpu as pltpu
from jax.experimental.pallas import tpu_sc as plsc

_B = 16384
_NEWS_D = 768
_CAT_D = 128
_FEAT = 1024
_OUT_D = 256

_NC = 2
_NS = 16
_NW = _NC * _NS

_SLICES = (8192, 8192)
_BM = 512


def _make_sc_gather(lo, sb):
    bpw = sb // _NW
    nch = -(-bpw // 128)
    ch = bpw // nch

    @functools.partial(
        pl.kernel,
        out_type=[
            jax.ShapeDtypeStruct((sb, _NEWS_D), jnp.float32),
            jax.ShapeDtypeStruct((sb, _CAT_D), jnp.float32),
            jax.ShapeDtypeStruct((sb, _CAT_D), jnp.float32),
        ],
        mesh=plsc.VectorSubcoreMesh(core_axis_name="c", subcore_axis_name="s"),
        scratch_types=[
            pltpu.VMEM((bpw,), jnp.int32),
            pltpu.VMEM((bpw,), jnp.int32),
            pltpu.VMEM((bpw,), jnp.int32),
            pltpu.VMEM((ch, _NEWS_D), jnp.float32),
            pltpu.VMEM((ch, _CAT_D), jnp.float32),
            pltpu.SemaphoreType.DMA,
            pltpu.SemaphoreType.DMA,
        ],
    )
    def sc_gather(news_table_h, cat_table_h, sub_table_h, nid_h, cid_h, sid_h,
                  news_out, cat_out, sub_out,
                  nid_v, cid_v, sid_v, nb, cb, nsem, csem):
        wid = lax.axis_index("s") * _NC + lax.axis_index("c")
        base = lo + wid * bpw
        obase = wid * bpw
        pltpu.sync_copy(nid_h.at[pl.ds(base, bpw)], nid_v)
        pltpu.sync_copy(cid_h.at[pl.ds(base, bpw)], cid_v)
        pltpu.sync_copy(sid_h.at[pl.ds(base, bpw)], sid_v)

        def nfire(c):
            pltpu.async_copy(
                news_table_h.at[nid_v.at[pl.ds(c * ch, ch)]], nb, nsem)

        def small(table_h, idx_v, out_h, c):
            pltpu.async_copy(
                table_h.at[idx_v.at[pl.ds(c * ch, ch)]], cb, csem)
            pltpu.make_async_copy(
                table_h.at[idx_v.at[pl.ds(c * ch, ch)]], cb, csem).wait()
            pltpu.sync_copy(cb, out_h.at[pl.ds(obase + c * ch, ch)])

        for c in range(nch):
            nfire(c)
            small(cat_table_h, cid_v, cat_out, c)
            small(sub_table_h, sid_v, sub_out, c)
            pltpu.make_async_copy(
                news_table_h.at[nid_v.at[pl.ds(c * ch, ch)]], nb, nsem).wait()
            pltpu.sync_copy(nb, news_out.at[pl.ds(obase + c * ch, ch)])

    return sc_gather


_SC_GATHERS = []
_OFFS = []
_off = 0
for _sb in _SLICES:
    _SC_GATHERS.append(_make_sc_gather(_off, _sb))
    _OFFS.append(_off)
    _off += _sb


def _gelu_tanh(x):
    c0 = math.sqrt(2.0 / math.pi)
    return 0.5 * x * (1.0 + jnp.tanh(c0 * (x + 0.044715 * x * x * x)))


def _tc_compute(n_ref, c_ref, s_ref, w_ref, b_ref, o_ref):
    w = w_ref[...]
    acc = jnp.dot(n_ref[...], w[:_NEWS_D], preferred_element_type=jnp.float32)
    acc = acc + jnp.dot(c_ref[...], w[_NEWS_D:_NEWS_D + _CAT_D],
                        preferred_element_type=jnp.float32)
    acc = acc + jnp.dot(s_ref[...], w[_NEWS_D + _CAT_D:],
                        preferred_element_type=jnp.float32)
    acc = acc + b_ref[...]
    o_ref[...] = _gelu_tanh(acc)


def _tc_body_first(n_ref, c_ref, s_ref, w_ref, b_ref, o_ref):
    _tc_compute(n_ref, c_ref, s_ref, w_ref, b_ref, o_ref)


def _tc_body_next(p_ref, n_ref, c_ref, s_ref, w_ref, b_ref, o_ref):
    del p_ref
    _tc_compute(n_ref, c_ref, s_ref, w_ref, b_ref, o_ref)


def _tc_fused(lo, sb, prev, news_g, cat_g, sub_g, W, b2):
    blk0 = lo // _BM
    data_specs = [
        pl.BlockSpec((_BM, _NEWS_D), lambda i: (i, 0)),
        pl.BlockSpec((_BM, _CAT_D), lambda i: (i, 0)),
        pl.BlockSpec((_BM, _CAT_D), lambda i: (i, 0)),
        pl.BlockSpec((_FEAT, _OUT_D), lambda i: (0, 0)),
        pl.BlockSpec((1, _OUT_D), lambda i: (0, 0)),
    ]
    if prev is None:
        body, in_specs, aliases, args = (
            _tc_body_first, data_specs, {}, ())
    else:
        body = _tc_body_next
        in_specs = [pl.BlockSpec(memory_space=pl.ANY)] + data_specs
        aliases = {0: 0}
        args = (prev,)
    return pl.pallas_call(
        body,
        grid=(sb // _BM,),
        in_specs=in_specs,
        out_specs=pl.BlockSpec((_BM, _OUT_D), lambda i, _b=blk0: (_b + i, 0)),
        out_shape=jax.ShapeDtypeStruct((_B, _OUT_D), jnp.float32),
        input_output_aliases=aliases,
        compiler_params=pltpu.CompilerParams(
            dimension_semantics=("arbitrary",)),
    )(*args, news_g, cat_g, sub_g, W, b2)


def kernel(news_ids, news_categ, news_subcateg, news_table, cat_table,
           subcat_table, W, b):
    nid = news_ids.astype(jnp.int32)
    cid = news_categ.astype(jnp.int32)
    sid = news_subcateg.astype(jnp.int32)
    b2 = b.reshape(1, _OUT_D)
    gathered = [
        sc(news_table, cat_table, subcat_table, nid, cid, sid)
        for sc in _SC_GATHERS
    ]
    out = None
    for s, sb in enumerate(_SLICES):
        news_g, cat_g, sub_g = gathered[s]
        out = _tc_fused(_OFFS[s], sb, out, news_g, cat_g, sub_g, W, b2)
    return out


# speedup vs baseline: 1.0666x; 1.0449x over previous
import functools
import math

import jax
import jax.numpy as jnp
from jax import lax
from jax.experimental import pallas as pl
from jax.experimental.pallas import tpu as pltpu
from jax.experimental.pallas import tpu_sc as plsc

_B = 16384
_NEWS_D = 768
_CAT_D = 128
_FEAT = 1024
_OUT_D = 256

_NC = 2
_NS = 16
_NW = _NC * _NS

_SLICES = (8192, 8192)
_BM = 1024


def _make_sc_gather(lo, sb):
    bpw = sb // _NW
    nch = -(-bpw // 128)
    ch = bpw // nch

    @functools.partial(
        pl.kernel,
        out_type=[
            jax.ShapeDtypeStruct((sb, _NEWS_D), jnp.float32),
            jax.ShapeDtypeStruct((sb, _CAT_D), jnp.float32),
            jax.ShapeDtypeStruct((sb, _CAT_D), jnp.float32),
        ],
        mesh=plsc.VectorSubcoreMesh(core_axis_name="c", subcore_axis_name="s"),
        scratch_types=[
            pltpu.VMEM((bpw,), jnp.int32),
            pltpu.VMEM((bpw,), jnp.int32),
            pltpu.VMEM((bpw,), jnp.int32),
            pltpu.VMEM((ch, _NEWS_D), jnp.float32),
            pltpu.VMEM((ch, _CAT_D), jnp.float32),
            pltpu.SemaphoreType.DMA,
            pltpu.SemaphoreType.DMA,
        ],
    )
    def sc_gather(news_table_h, cat_table_h, sub_table_h, nid_h, cid_h, sid_h,
                  news_out, cat_out, sub_out,
                  nid_v, cid_v, sid_v, nb, cb, nsem, csem):
        wid = lax.axis_index("s") * _NC + lax.axis_index("c")
        base = lo + wid * bpw
        obase = wid * bpw
        pltpu.sync_copy(nid_h.at[pl.ds(base, bpw)], nid_v)
        pltpu.sync_copy(cid_h.at[pl.ds(base, bpw)], cid_v)
        pltpu.sync_copy(sid_h.at[pl.ds(base, bpw)], sid_v)

        def nfire(c):
            pltpu.async_copy(
                news_table_h.at[nid_v.at[pl.ds(c * ch, ch)]], nb, nsem)

        def small(table_h, idx_v, out_h, c):
            pltpu.async_copy(
                table_h.at[idx_v.at[pl.ds(c * ch, ch)]], cb, csem)
            pltpu.make_async_copy(
                table_h.at[idx_v.at[pl.ds(c * ch, ch)]], cb, csem).wait()
            pltpu.sync_copy(cb, out_h.at[pl.ds(obase + c * ch, ch)])

        for c in range(nch):
            nfire(c)
            small(cat_table_h, cid_v, cat_out, c)
            small(sub_table_h, sid_v, sub_out, c)
            pltpu.make_async_copy(
                news_table_h.at[nid_v.at[pl.ds(c * ch, ch)]], nb, nsem).wait()
            pltpu.sync_copy(nb, news_out.at[pl.ds(obase + c * ch, ch)])

    return sc_gather


_SC_GATHERS = []
_OFFS = []
_off = 0
for _sb in _SLICES:
    _SC_GATHERS.append(_make_sc_gather(_off, _sb))
    _OFFS.append(_off)
    _off += _sb


def _gelu_tanh(x):
    c0 = math.sqrt(2.0 / math.pi)
    return 0.5 * x * (1.0 + jnp.tanh(c0 * (x + 0.044715 * x * x * x)))


def _tc_compute(n_ref, c_ref, s_ref, w_ref, b_ref, o_ref):
    w = w_ref[...]
    acc = jnp.dot(n_ref[...], w[:_NEWS_D], preferred_element_type=jnp.float32)
    acc = acc + jnp.dot(c_ref[...], w[_NEWS_D:_NEWS_D + _CAT_D],
                        preferred_element_type=jnp.float32)
    acc = acc + jnp.dot(s_ref[...], w[_NEWS_D + _CAT_D:],
                        preferred_element_type=jnp.float32)
    acc = acc + b_ref[...]
    o_ref[...] = _gelu_tanh(acc)


def _tc_body_first(n_ref, c_ref, s_ref, w_ref, b_ref, o_ref):
    _tc_compute(n_ref, c_ref, s_ref, w_ref, b_ref, o_ref)


def _tc_body_next(p_ref, n_ref, c_ref, s_ref, w_ref, b_ref, o_ref):
    del p_ref
    _tc_compute(n_ref, c_ref, s_ref, w_ref, b_ref, o_ref)


def _tc_fused(lo, sb, prev, news_g, cat_g, sub_g, W, b2):
    blk0 = lo // _BM
    data_specs = [
        pl.BlockSpec((_BM, _NEWS_D), lambda i: (i, 0)),
        pl.BlockSpec((_BM, _CAT_D), lambda i: (i, 0)),
        pl.BlockSpec((_BM, _CAT_D), lambda i: (i, 0)),
        pl.BlockSpec((_FEAT, _OUT_D), lambda i: (0, 0)),
        pl.BlockSpec((1, _OUT_D), lambda i: (0, 0)),
    ]
    if prev is None:
        body, in_specs, aliases, args = (
            _tc_body_first, data_specs, {}, ())
    else:
        body = _tc_body_next
        in_specs = [pl.BlockSpec(memory_space=pl.ANY)] + data_specs
        aliases = {0: 0}
        args = (prev,)
    return pl.pallas_call(
        body,
        grid=(sb // _BM,),
        in_specs=in_specs,
        out_specs=pl.BlockSpec((_BM, _OUT_D), lambda i, _b=blk0: (_b + i, 0)),
        out_shape=jax.ShapeDtypeStruct((_B, _OUT_D), jnp.float32),
        input_output_aliases=aliases,
        compiler_params=pltpu.CompilerParams(
            dimension_semantics=("arbitrary",)),
    )(*args, news_g, cat_g, sub_g, W, b2)


def kernel(news_ids, news_categ, news_subcateg, news_table, cat_table,
           subcat_table, W, b):
    nid = news_ids.astype(jnp.int32)
    cid = news_categ.astype(jnp.int32)
    sid = news_subcateg.astype(jnp.int32)
    b2 = b.reshape(1, _OUT_D)
    gathered = [
        sc(news_table, cat_table, subcat_table, nid, cid, sid)
        for sc in _SC_GATHERS
    ]
    out = None
    for s, sb in enumerate(_SLICES):
        news_g, cat_g, sub_g = gathered[s]
        out = _tc_fused(_OFFS[s], sb, out, news_g, cat_g, sub_g, W, b2)
    return out


# speedup vs baseline: 1.0800x; 1.0125x over previous
import functools
import math

import jax
import jax.numpy as jnp
from jax import lax
from jax.experimental import pallas as pl
from jax.experimental.pallas import tpu as pltpu
from jax.experimental.pallas import tpu_sc as plsc

_B = 16384
_NEWS_D = 768
_CAT_D = 128
_FEAT = 1024
_OUT_D = 256

_NC = 2
_NS = 16
_NW = _NC * _NS

_SLICES = (8192, 8192)
_BM = 2048


def _make_sc_gather(lo, sb):
    bpw = sb // _NW
    nch = -(-bpw // 128)
    ch = bpw // nch

    @functools.partial(
        pl.kernel,
        out_type=[
            jax.ShapeDtypeStruct((sb, _NEWS_D), jnp.float32),
            jax.ShapeDtypeStruct((sb, _CAT_D), jnp.float32),
            jax.ShapeDtypeStruct((sb, _CAT_D), jnp.float32),
        ],
        mesh=plsc.VectorSubcoreMesh(core_axis_name="c", subcore_axis_name="s"),
        scratch_types=[
            pltpu.VMEM((bpw,), jnp.int32),
            pltpu.VMEM((bpw,), jnp.int32),
            pltpu.VMEM((bpw,), jnp.int32),
            pltpu.VMEM((ch, _NEWS_D), jnp.float32),
            pltpu.VMEM((ch, _CAT_D), jnp.float32),
            pltpu.SemaphoreType.DMA,
            pltpu.SemaphoreType.DMA,
        ],
    )
    def sc_gather(news_table_h, cat_table_h, sub_table_h, nid_h, cid_h, sid_h,
                  news_out, cat_out, sub_out,
                  nid_v, cid_v, sid_v, nb, cb, nsem, csem):
        wid = lax.axis_index("s") * _NC + lax.axis_index("c")
        base = lo + wid * bpw
        obase = wid * bpw
        pltpu.sync_copy(nid_h.at[pl.ds(base, bpw)], nid_v)
        pltpu.sync_copy(cid_h.at[pl.ds(base, bpw)], cid_v)
        pltpu.sync_copy(sid_h.at[pl.ds(base, bpw)], sid_v)

        def nfire(c):
            pltpu.async_copy(
                news_table_h.at[nid_v.at[pl.ds(c * ch, ch)]], nb, nsem)

        def small(table_h, idx_v, out_h, c):
            pltpu.async_copy(
                table_h.at[idx_v.at[pl.ds(c * ch, ch)]], cb, csem)
            pltpu.make_async_copy(
                table_h.at[idx_v.at[pl.ds(c * ch, ch)]], cb, csem).wait()
            pltpu.sync_copy(cb, out_h.at[pl.ds(obase + c * ch, ch)])

        for c in range(nch):
            nfire(c)
            small(cat_table_h, cid_v, cat_out, c)
            small(sub_table_h, sid_v, sub_out, c)
            pltpu.make_async_copy(
                news_table_h.at[nid_v.at[pl.ds(c * ch, ch)]], nb, nsem).wait()
            pltpu.sync_copy(nb, news_out.at[pl.ds(obase + c * ch, ch)])

    return sc_gather


_SC_GATHERS = []
_OFFS = []
_off = 0
for _sb in _SLICES:
    _SC_GATHERS.append(_make_sc_gather(_off, _sb))
    _OFFS.append(_off)
    _off += _sb


def _gelu_tanh(x):
    c0 = math.sqrt(2.0 / math.pi)
    return 0.5 * x * (1.0 + jnp.tanh(c0 * (x + 0.044715 * x * x * x)))


def _tc_compute(n_ref, c_ref, s_ref, w_ref, b_ref, o_ref):
    w = w_ref[...]
    acc = jnp.dot(n_ref[...], w[:_NEWS_D], preferred_element_type=jnp.float32)
    acc = acc + jnp.dot(c_ref[...], w[_NEWS_D:_NEWS_D + _CAT_D],
                        preferred_element_type=jnp.float32)
    acc = acc + jnp.dot(s_ref[...], w[_NEWS_D + _CAT_D:],
                        preferred_element_type=jnp.float32)
    acc = acc + b_ref[...]
    o_ref[...] = _gelu_tanh(acc)


def _tc_body_first(n_ref, c_ref, s_ref, w_ref, b_ref, o_ref):
    _tc_compute(n_ref, c_ref, s_ref, w_ref, b_ref, o_ref)


def _tc_body_next(p_ref, n_ref, c_ref, s_ref, w_ref, b_ref, o_ref):
    del p_ref
    _tc_compute(n_ref, c_ref, s_ref, w_ref, b_ref, o_ref)


def _tc_fused(lo, sb, prev, news_g, cat_g, sub_g, W, b2):
    blk0 = lo // _BM
    data_specs = [
        pl.BlockSpec((_BM, _NEWS_D), lambda i: (i, 0)),
        pl.BlockSpec((_BM, _CAT_D), lambda i: (i, 0)),
        pl.BlockSpec((_BM, _CAT_D), lambda i: (i, 0)),
        pl.BlockSpec((_FEAT, _OUT_D), lambda i: (0, 0)),
        pl.BlockSpec((1, _OUT_D), lambda i: (0, 0)),
    ]
    if prev is None:
        body, in_specs, aliases, args = (
            _tc_body_first, data_specs, {}, ())
    else:
        body = _tc_body_next
        in_specs = [pl.BlockSpec(memory_space=pl.ANY)] + data_specs
        aliases = {0: 0}
        args = (prev,)
    return pl.pallas_call(
        body,
        grid=(sb // _BM,),
        in_specs=in_specs,
        out_specs=pl.BlockSpec((_BM, _OUT_D), lambda i, _b=blk0: (_b + i, 0)),
        out_shape=jax.ShapeDtypeStruct((_B, _OUT_D), jnp.float32),
        input_output_aliases=aliases,
        compiler_params=pltpu.CompilerParams(
            dimension_semantics=("arbitrary",)),
    )(*args, news_g, cat_g, sub_g, W, b2)


def kernel(news_ids, news_categ, news_subcateg, news_table, cat_table,
           subcat_table, W, b):
    nid = news_ids.astype(jnp.int32)
    cid = news_categ.astype(jnp.int32)
    sid = news_subcateg.astype(jnp.int32)
    b2 = b.reshape(1, _OUT_D)
    gathered = [
        sc(news_table, cat_table, subcat_table, nid, cid, sid)
        for sc in _SC_GATHERS
    ]
    out = None
    for s, sb in enumerate(_SLICES):
        news_g, cat_g, sub_g = gathered[s]
        out = _tc_fused(_OFFS[s], sb, out, news_g, cat_g, sub_g, W, b2)
    return out


# speedup vs baseline: 1.1084x; 1.0263x over previous
import functools
import math

import jax
import jax.numpy as jnp
from jax import lax
from jax.experimental import pallas as pl
from jax.experimental.pallas import tpu as pltpu
from jax.experimental.pallas import tpu_sc as plsc

_B = 16384
_NEWS_D = 768
_CAT_D = 128
_FEAT = 1024
_OUT_D = 256

_NC = 2
_NS = 16
_NW = _NC * _NS

_SLICES = (8192, 8192)
_BM = 4096


def _make_sc_gather(lo, sb):
    bpw = sb // _NW
    nch = -(-bpw // 128)
    ch = bpw // nch

    @functools.partial(
        pl.kernel,
        out_type=[
            jax.ShapeDtypeStruct((sb, _NEWS_D), jnp.float32),
            jax.ShapeDtypeStruct((sb, _CAT_D), jnp.float32),
            jax.ShapeDtypeStruct((sb, _CAT_D), jnp.float32),
        ],
        mesh=plsc.VectorSubcoreMesh(core_axis_name="c", subcore_axis_name="s"),
        scratch_types=[
            pltpu.VMEM((bpw,), jnp.int32),
            pltpu.VMEM((bpw,), jnp.int32),
            pltpu.VMEM((bpw,), jnp.int32),
            pltpu.VMEM((ch, _NEWS_D), jnp.float32),
            pltpu.VMEM((ch, _CAT_D), jnp.float32),
            pltpu.SemaphoreType.DMA,
            pltpu.SemaphoreType.DMA,
        ],
    )
    def sc_gather(news_table_h, cat_table_h, sub_table_h, nid_h, cid_h, sid_h,
                  news_out, cat_out, sub_out,
                  nid_v, cid_v, sid_v, nb, cb, nsem, csem):
        wid = lax.axis_index("s") * _NC + lax.axis_index("c")
        base = lo + wid * bpw
        obase = wid * bpw
        pltpu.sync_copy(nid_h.at[pl.ds(base, bpw)], nid_v)
        pltpu.sync_copy(cid_h.at[pl.ds(base, bpw)], cid_v)
        pltpu.sync_copy(sid_h.at[pl.ds(base, bpw)], sid_v)

        def nfire(c):
            pltpu.async_copy(
                news_table_h.at[nid_v.at[pl.ds(c * ch, ch)]], nb, nsem)

        def small(table_h, idx_v, out_h, c):
            pltpu.async_copy(
                table_h.at[idx_v.at[pl.ds(c * ch, ch)]], cb, csem)
            pltpu.make_async_copy(
                table_h.at[idx_v.at[pl.ds(c * ch, ch)]], cb, csem).wait()
            pltpu.sync_copy(cb, out_h.at[pl.ds(obase + c * ch, ch)])

        for c in range(nch):
            nfire(c)
            small(cat_table_h, cid_v, cat_out, c)
            small(sub_table_h, sid_v, sub_out, c)
            pltpu.make_async_copy(
                news_table_h.at[nid_v.at[pl.ds(c * ch, ch)]], nb, nsem).wait()
            pltpu.sync_copy(nb, news_out.at[pl.ds(obase + c * ch, ch)])

    return sc_gather


_SC_GATHERS = []
_OFFS = []
_off = 0
for _sb in _SLICES:
    _SC_GATHERS.append(_make_sc_gather(_off, _sb))
    _OFFS.append(_off)
    _off += _sb


def _gelu_tanh(x):
    c0 = math.sqrt(2.0 / math.pi)
    return 0.5 * x * (1.0 + jnp.tanh(c0 * (x + 0.044715 * x * x * x)))


def _tc_compute(n_ref, c_ref, s_ref, w_ref, b_ref, o_ref):
    w = w_ref[...]
    acc = jnp.dot(n_ref[...], w[:_NEWS_D], preferred_element_type=jnp.float32)
    acc = acc + jnp.dot(c_ref[...], w[_NEWS_D:_NEWS_D + _CAT_D],
                        preferred_element_type=jnp.float32)
    acc = acc + jnp.dot(s_ref[...], w[_NEWS_D + _CAT_D:],
                        preferred_element_type=jnp.float32)
    acc = acc + b_ref[...]
    o_ref[...] = _gelu_tanh(acc)


def _tc_body_first(n_ref, c_ref, s_ref, w_ref, b_ref, o_ref):
    _tc_compute(n_ref, c_ref, s_ref, w_ref, b_ref, o_ref)


def _tc_body_next(p_ref, n_ref, c_ref, s_ref, w_ref, b_ref, o_ref):
    del p_ref
    _tc_compute(n_ref, c_ref, s_ref, w_ref, b_ref, o_ref)


def _tc_fused(lo, sb, prev, news_g, cat_g, sub_g, W, b2):
    blk0 = lo // _BM
    data_specs = [
        pl.BlockSpec((_BM, _NEWS_D), lambda i: (i, 0)),
        pl.BlockSpec((_BM, _CAT_D), lambda i: (i, 0)),
        pl.BlockSpec((_BM, _CAT_D), lambda i: (i, 0)),
        pl.BlockSpec((_FEAT, _OUT_D), lambda i: (0, 0)),
        pl.BlockSpec((1, _OUT_D), lambda i: (0, 0)),
    ]
    if prev is None:
        body, in_specs, aliases, args = (
            _tc_body_first, data_specs, {}, ())
    else:
        body = _tc_body_next
        in_specs = [pl.BlockSpec(memory_space=pl.ANY)] + data_specs
        aliases = {0: 0}
        args = (prev,)
    return pl.pallas_call(
        body,
        grid=(sb // _BM,),
        in_specs=in_specs,
        out_specs=pl.BlockSpec((_BM, _OUT_D), lambda i, _b=blk0: (_b + i, 0)),
        out_shape=jax.ShapeDtypeStruct((_B, _OUT_D), jnp.float32),
        input_output_aliases=aliases,
        compiler_params=pltpu.CompilerParams(
            dimension_semantics=("arbitrary",)),
    )(*args, news_g, cat_g, sub_g, W, b2)


def kernel(news_ids, news_categ, news_subcateg, news_table, cat_table,
           subcat_table, W, b):
    nid = news_ids.astype(jnp.int32)
    cid = news_categ.astype(jnp.int32)
    sid = news_subcateg.astype(jnp.int32)
    b2 = b.reshape(1, _OUT_D)
    gathered = [
        sc(news_table, cat_table, subcat_table, nid, cid, sid)
        for sc in _SC_GATHERS
    ]
    out = None
    for s, sb in enumerate(_SLICES):
        news_g, cat_g, sub_g = gathered[s]
        out = _tc_fused(_OFFS[s], sb, out, news_g, cat_g, sub_g, W, b2)
    return out
